# Initial kernel scaffold; baseline (speedup 1.0000x reference)
import jax
import jax.numpy as jnp
from jax.experimental import pallas as pl


def kernel(vertices, visible, label, weight):
    raise NotImplementedError("write your pallas kernel here")


# speedup vs baseline: 3.2742x; 3.2742x over previous
import jax
import jax.numpy as jnp
from jax.experimental import pallas as pl
from jax.experimental.pallas import tpu as pltpu

N_CLASSES = 32
MAX_N = 512
MESH_DIM = 128
BATCH = 16
MOMENTUM = 0.999
_EPS = 1e-12

_VB = 128


def _body(label_ref, vertices_ref, visible_ref, weight_ref, out_ref):
    lab = label_ref[...]
    vert = vertices_ref[...]
    vis = visible_ref[...]
    w = weight_ref[...]

    kk = jax.lax.broadcasted_iota(jnp.int32, (BATCH, N_CLASSES), 1)
    onehot = (lab == kk).astype(jnp.float32)

    scaled = vert * vis[:, :, None]
    scaled2 = scaled.reshape(BATCH, _VB * MESH_DIM)
    upd = jax.lax.dot_general(
        onehot, scaled2, (((0,), (0,)), ((), ())),
        preferred_element_type=jnp.float32,
    ).reshape(N_CLASSES, _VB, MESH_DIM)
    cnt = jax.lax.dot_general(
        onehot, vis, (((0,), (0,)), ((), ())),
        preferred_element_type=jnp.float32,
    )

    upd = upd / jnp.maximum(cnt, 1.0)[:, :, None]
    ss = jnp.sum(upd * upd, axis=-1, keepdims=True)
    nu = upd / jnp.maximum(jnp.sqrt(ss), _EPS)

    comb = MOMENTUM * w + (1.0 - MOMENTUM) * nu
    ss2 = jnp.sum(comb * comb, axis=-1, keepdims=True)
    out_ref[...] = comb / jnp.maximum(jnp.sqrt(ss2), _EPS)


def kernel(vertices, visible, label, weight):
    lab2 = label.astype(jnp.int32).reshape(BATCH, 1)
    grid = MAX_N // _VB
    return pl.pallas_call(
        _body,
        grid=(grid,),
        in_specs=[
            pl.BlockSpec((BATCH, 1), lambda i: (0, 0)),
            pl.BlockSpec((BATCH, _VB, MESH_DIM), lambda i: (0, i, 0)),
            pl.BlockSpec((BATCH, _VB), lambda i: (0, i)),
            pl.BlockSpec((N_CLASSES, _VB, MESH_DIM), lambda i: (0, i, 0)),
        ],
        out_specs=pl.BlockSpec((N_CLASSES, _VB, MESH_DIM), lambda i: (0, i, 0)),
        out_shape=jax.ShapeDtypeStruct((N_CLASSES, MAX_N, MESH_DIM), jnp.float32),
    )(lab2, vertices, visible, weight)
